# baseline (speedup 1.0000x reference)
import functools

import jax
import jax.numpy as jnp
from jax import lax
from jax.experimental import pallas as pl
from jax.experimental.pallas import tpu as pltpu
from jax.experimental.pallas import tpu_sc as plsc

N = 10000
D = 128
E = 320000

NC = 2
NS = 16
L = 16
NW = NC * NS
EPW = E // NW
CW = 40
NCH = EPW // CW
NBUF = 4
NGRP = (NCH + NBUF) // NBUF
RPT = N // NS

_mesh = plsc.VectorSubcoreMesh(core_axis_name="c", subcore_axis_name="s")


@functools.partial(
    pl.kernel,
    out_type=jax.ShapeDtypeStruct((NC, NS, RPT, D), jnp.float32),
    mesh=_mesh,
    scratch_types=(
        [pltpu.VMEM((2, CW), jnp.int32)] * NBUF
        + [pltpu.VMEM((CW, L), jnp.float32)] * NBUF
        + [pltpu.VMEM((CW, D), jnp.float32)] * NBUF
        + [pltpu.VMEM_SHARED((N, D), jnp.float32)]
        + [pltpu.SemaphoreType.DMA] * (3 * NBUF)
    ),
)
def _sc_edge_agg(hn_hbm, meta_hbm, w_hbm, zrows_hbm, out_hbm,
                 i0, i1, i2, i3, w0, w1, w2, w3, r0, r1, r2, r3, acc,
                 m0, m1, m2, m3, g0, g1, g2, g3, s0, s1, s2, s3):
    idx_b = [i0, i1, i2, i3]
    w_b = [w0, w1, w2, w3]
    rows_b = [r0, r1, r2, r3]
    sem_m = [m0, m1, m2, m3]
    sem_g = [g0, g1, g2, g3]
    sem_s = [s0, s1, s2, s3]
    c = lax.axis_index("c")
    s = lax.axis_index("s")
    wid = s * NC + c

    pltpu.sync_copy(zrows_hbm, acc.at[pl.ds(s * RPT, RPT)])
    plsc.subcore_barrier()

    def start_meta(t, b):
        pltpu.async_copy(meta_hbm.at[wid, t], idx_b[b], sem_m[b])
        pltpu.async_copy(w_hbm.at[wid, t], w_b[b], sem_m[b])

    def wait_meta(t, b):
        pltpu.make_async_copy(meta_hbm.at[wid, t], idx_b[b], sem_m[b]).wait()
        pltpu.make_async_copy(w_hbm.at[wid, t], w_b[b], sem_m[b]).wait()

    def start_gather(b):
        pltpu.async_copy(hn_hbm.at[idx_b[b].at[0]], rows_b[b], sem_g[b])

    def wait_gather(b):
        pltpu.make_async_copy(hn_hbm.at[idx_b[b].at[0]], rows_b[b],
                              sem_g[b]).wait()

    def start_scatter(b):
        pltpu.async_copy(rows_b[b], acc.at[idx_b[b].at[1]], sem_s[b],
                         add=True)

    def wait_scatter(b):
        pltpu.make_async_copy(rows_b[b], acc.at[idx_b[b].at[1]],
                              sem_s[b]).wait()

    start_meta(0, 0)
    start_meta(1, 1)
    wait_meta(0, 0)
    start_gather(0)

    def group(g, carry):
        for b in range(NBUF):
            t = g * NBUF + b

            @pl.when(t >= 3)
            def _():
                wait_scatter((b + 1) % NBUF)

            @pl.when(t + 2 < NCH)
            def _():
                start_meta(t + 2, (b + 2) % NBUF)

            @pl.when(t + 1 < NCH)
            def _():
                wait_meta(t + 1, (b + 1) % NBUF)
                start_gather((b + 1) % NBUF)

            @pl.when(t < NCH)
            def _():
                wait_gather(b)

                @plsc.parallel_loop(0, CW, unroll=4)
                def _edge(e):
                    wb = w_b[b][e]
                    for k in range(D // L):
                        rows_b[b][e, pl.ds(k * L, L)] = (
                            rows_b[b][e, pl.ds(k * L, L)] * wb)

                start_scatter(b)
        return carry

    lax.fori_loop(0, NGRP, group, 0)
    wait_scatter((NCH - 1) % NBUF)
    plsc.subcore_barrier()

    pltpu.sync_copy(acc.at[pl.ds(s * RPT, RPT)], out_hbm.at[c, s])


_BLK = 2000
_GRID = N // _BLK


def _mm_first(x_ref, sk_ref, nk_ref, hs_ref, hn_ref):
    x = x_ref[...]
    hs_ref[...] = jnp.dot(x, sk_ref[...], preferred_element_type=jnp.float32)
    hn_ref[...] = jnp.dot(x, nk_ref[...], preferred_element_type=jnp.float32)


def _mm_mid(hs_ref, p_ref, b_ref, sk_ref, nk_ref, hs_ref_o, hn_ref_o):
    h = jnp.maximum(hs_ref[...] + p_ref[0] + p_ref[1] + b_ref[...], 0.0)
    hs_ref_o[...] = jnp.dot(h, sk_ref[...], preferred_element_type=jnp.float32)
    hn_ref_o[...] = jnp.dot(h, nk_ref[...], preferred_element_type=jnp.float32)


def _final(hs_ref, p_ref, b_ref, o_ref):
    o_ref[...] = jnp.maximum(hs_ref[...] + p_ref[0] + p_ref[1] + b_ref[...], 0.0)


def _row_spec():
    return pl.BlockSpec((_BLK, D), lambda i: (i, 0))


def _full_spec(shape):
    nd = len(shape)
    return pl.BlockSpec(shape, lambda i: (0,) * nd)


_mm_first_call = pl.pallas_call(
    _mm_first,
    grid=(_GRID,),
    in_specs=[_row_spec(), _full_spec((D, D)), _full_spec((D, D))],
    out_specs=[_row_spec(), _row_spec()],
    out_shape=[jax.ShapeDtypeStruct((N, D), jnp.float32)] * 2,
)

_mm_mid_call = pl.pallas_call(
    _mm_mid,
    grid=(_GRID,),
    in_specs=[_row_spec(), pl.BlockSpec((NC, _BLK, D), lambda i: (0, i, 0)),
              _full_spec((1, D)), _full_spec((D, D)), _full_spec((D, D))],
    out_specs=[_row_spec(), _row_spec()],
    out_shape=[jax.ShapeDtypeStruct((N, D), jnp.float32)] * 2,
)

_final_call = pl.pallas_call(
    _final,
    grid=(_GRID,),
    in_specs=[_row_spec(), pl.BlockSpec((NC, _BLK, D), lambda i: (0, i, 0)),
              _full_spec((1, D))],
    out_specs=_row_spec(),
    out_shape=jax.ShapeDtypeStruct((N, D), jnp.float32),
)


def kernel(x, edge_index, edge_weight, self_kernel_0, neighbor_kernel_0,
           bias_0, self_kernel_1, neighbor_kernel_1, bias_1):
    col = edge_index[1].reshape(NW, NCH, 1, CW)
    row = edge_index[0].reshape(NW, NCH, 1, CW)
    meta = jnp.concatenate([col, row], axis=2)
    w = jnp.broadcast_to(edge_weight[:, None], (E, L)).reshape(NW, NCH, CW, L)
    zrows = jnp.zeros((RPT, D), jnp.float32)
    b0 = bias_0.reshape(1, D)
    b1 = bias_1.reshape(1, D)

    hs0, hn0 = _mm_first_call(x, self_kernel_0, neighbor_kernel_0)
    p0 = _sc_edge_agg(hn0, meta, w, zrows).reshape(NC, N, D)
    hs1, hn1 = _mm_mid_call(hs0, p0, b0, self_kernel_1, neighbor_kernel_1)
    p1 = _sc_edge_agg(hn1, meta, w, zrows).reshape(NC, N, D)
    return _final_call(hs1, p1, b1)


# speedup vs baseline: 1.0046x; 1.0046x over previous
import functools

import jax
import jax.numpy as jnp
import numpy as np
from jax import lax
from jax.experimental import pallas as pl
from jax.experimental.pallas import tpu as pltpu
from jax.experimental.pallas import tpu_sc as plsc

N = 10000
D = 128
E = 320000

NC = 2
NS = 16
L = 16
NW = NC * NS
EPW = E // NW
CW = 40
NCH = EPW // CW
NBUF = 4
NGRP = (NCH + 3 + NBUF - 1) // NBUF
RPT = N // NS

_mesh = plsc.VectorSubcoreMesh(core_axis_name="c", subcore_axis_name="s")


@functools.partial(
    pl.kernel,
    out_type=jax.ShapeDtypeStruct((NC, NS, RPT, D), jnp.float32),
    mesh=_mesh,
    scratch_types=(
        [pltpu.VMEM((2, CW), jnp.int32)] * NBUF
        + [pltpu.VMEM((CW, L), jnp.float32)] * NBUF
        + [pltpu.VMEM((CW, D), jnp.float32)] * NBUF
        + [pltpu.VMEM_SHARED((N, D), jnp.float32)]
        + [pltpu.SemaphoreType.DMA] * (3 * NBUF)
    ),
)
def _sc_edge_agg(hn_hbm, meta_hbm, w_hbm, zrows_hbm, out_hbm,
                 i0, i1, i2, i3, w0, w1, w2, w3, r0, r1, r2, r3, acc,
                 m0, m1, m2, m3, g0, g1, g2, g3, s0, s1, s2, s3):
    idx_b = [i0, i1, i2, i3]
    w_b = [w0, w1, w2, w3]
    rows_b = [r0, r1, r2, r3]
    sem_m = [m0, m1, m2, m3]
    sem_g = [g0, g1, g2, g3]
    sem_s = [s0, s1, s2, s3]
    c = lax.axis_index("c")
    s = lax.axis_index("s")
    wid = s * NC + c

    def start_meta(t, b):
        pltpu.async_copy(meta_hbm.at[wid, t], idx_b[b], sem_m[b])
        pltpu.async_copy(w_hbm.at[wid, t], w_b[b], sem_m[b])

    def wait_meta(t, b):
        pltpu.make_async_copy(meta_hbm.at[wid, t], idx_b[b], sem_m[b]).wait()
        pltpu.make_async_copy(w_hbm.at[wid, t], w_b[b], sem_m[b]).wait()

    def start_gather(b):
        pltpu.async_copy(hn_hbm.at[idx_b[b].at[0]], rows_b[b], sem_g[b])

    def wait_gather(b):
        pltpu.make_async_copy(hn_hbm.at[idx_b[b].at[0]], rows_b[b],
                              sem_g[b]).wait()

    def start_scatter(b):
        pltpu.async_copy(rows_b[b], acc.at[idx_b[b].at[1]], sem_s[b],
                         add=True)

    def wait_scatter(b):
        pltpu.make_async_copy(rows_b[b], acc.at[idx_b[b].at[1]],
                              sem_s[b]).wait()

    zcp = pltpu.async_copy(zrows_hbm, acc.at[pl.ds(s * RPT, RPT)], sem_s[0])
    start_meta(0, 0)
    start_meta(1, 1)
    wait_meta(0, 0)
    start_gather(0)
    zcp.wait()
    plsc.subcore_barrier()

    def group(g, carry):
        for b in range(NBUF):
            t = g * NBUF + b

            @pl.when((t >= 3) & (t - 3 < NCH))
            def _():
                wait_scatter((b + 1) % NBUF)

            @pl.when(t + 2 < NCH)
            def _():
                start_meta(t + 2, (b + 2) % NBUF)

            @pl.when(t + 1 < NCH)
            def _():
                wait_meta(t + 1, (b + 1) % NBUF)
                start_gather((b + 1) % NBUF)

            @pl.when(t < NCH)
            def _():
                wait_gather(b)

                @plsc.parallel_loop(0, CW, unroll=4)
                def _edge(e):
                    wb = w_b[b][e]
                    for k in range(D // L):
                        rows_b[b][e, pl.ds(k * L, L)] = (
                            rows_b[b][e, pl.ds(k * L, L)] * wb)

                start_scatter(b)
        return carry

    lax.fori_loop(0, NGRP, group, 0)
    plsc.subcore_barrier()

    pltpu.sync_copy(acc.at[pl.ds(s * RPT, RPT)], out_hbm.at[c, s])


_BLK = 2000
_GRID = N // _BLK


def _mm_first(x_ref, sk_ref, nk_ref, hs_ref, hn_ref):
    x = x_ref[...]
    hs_ref[...] = jnp.dot(x, sk_ref[...], preferred_element_type=jnp.float32)
    hn_ref[...] = jnp.dot(x, nk_ref[...], preferred_element_type=jnp.float32)


def _mm_mid(hs_ref, p_ref, b_ref, sk_ref, nk_ref, hs_ref_o, hn_ref_o):
    h = jnp.maximum(hs_ref[...] + p_ref[0] + p_ref[1] + b_ref[...], 0.0)
    hs_ref_o[...] = jnp.dot(h, sk_ref[...], preferred_element_type=jnp.float32)
    hn_ref_o[...] = jnp.dot(h, nk_ref[...], preferred_element_type=jnp.float32)


def _final(hs_ref, p_ref, b_ref, o_ref):
    o_ref[...] = jnp.maximum(hs_ref[...] + p_ref[0] + p_ref[1] + b_ref[...], 0.0)


def _row_spec():
    return pl.BlockSpec((_BLK, D), lambda i: (i, 0))


def _full_spec(shape):
    nd = len(shape)
    return pl.BlockSpec(shape, lambda i: (0,) * nd)


_mm_first_call = pl.pallas_call(
    _mm_first,
    grid=(_GRID,),
    in_specs=[_row_spec(), _full_spec((D, D)), _full_spec((D, D))],
    out_specs=[_row_spec(), _row_spec()],
    out_shape=[jax.ShapeDtypeStruct((N, D), jnp.float32)] * 2,
)

_mm_mid_call = pl.pallas_call(
    _mm_mid,
    grid=(_GRID,),
    in_specs=[_row_spec(), pl.BlockSpec((NC, _BLK, D), lambda i: (0, i, 0)),
              _full_spec((1, D)), _full_spec((D, D)), _full_spec((D, D))],
    out_specs=[_row_spec(), _row_spec()],
    out_shape=[jax.ShapeDtypeStruct((N, D), jnp.float32)] * 2,
)

_final_call = pl.pallas_call(
    _final,
    grid=(_GRID,),
    in_specs=[_row_spec(), pl.BlockSpec((NC, _BLK, D), lambda i: (0, i, 0)),
              _full_spec((1, D))],
    out_specs=_row_spec(),
    out_shape=jax.ShapeDtypeStruct((N, D), jnp.float32),
)


def kernel(x, edge_index, edge_weight, self_kernel_0, neighbor_kernel_0,
           bias_0, self_kernel_1, neighbor_kernel_1, bias_1):
    nk0 = neighbor_kernel_0
    nk1 = neighbor_kernel_1
    col = edge_index[1].reshape(NW, NCH, 1, CW)
    row = edge_index[0].reshape(NW, NCH, 1, CW)
    meta = jnp.concatenate([col, row], axis=2)
    w = jnp.broadcast_to(edge_weight[:, None], (E, L)).reshape(NW, NCH, CW, L)
    zrows = jnp.zeros((RPT, D), jnp.float32)
    b0 = bias_0.reshape(1, D)
    b1 = bias_1.reshape(1, D)

    hs0, hn0 = _mm_first_call(x, self_kernel_0, nk0)
    p0 = _sc_edge_agg(hn0, meta, w, zrows).reshape(NC, N, D)
    hs1, hn1 = _mm_mid_call(hs0, p0, b0, self_kernel_1, nk1)
    p1 = _sc_edge_agg(hn1, meta, w, zrows).reshape(NC, N, D)
    return _final_call(hs1, p1, b1)
